# Initial kernel scaffold; baseline (speedup 1.0000x reference)
import jax
import jax.numpy as jnp
from jax.experimental import pallas as pl


def kernel(compound_x, compound_edge_index, compound_graph_ids, protein_x, protein_edge_index, protein_graph_ids, Wc1, bc1, Wc2, bc2, Wp1, bp1, Wp2, bp2, Wm1, bm1, Wm2, bm2):
    raise NotImplementedError("write your pallas kernel here")


# speedup vs baseline: 3.1544x; 3.1544x over previous
import jax
import jax.numpy as jnp
from jax import lax
from jax.experimental import pallas as pl
from jax.experimental.pallas import tpu as pltpu
from jax.experimental.pallas import tpu_sc as plsc

F32 = jnp.float32
I32 = jnp.int32

NC, EC = 50000, 800000
NP2, EP = 10000, 320000
NG = 1024
C_IN, P_IN, HID = 74, 128, 128

NCP = 50176
NPP = 10240
BP = 1040
ECP = 16 * 98 * 512
EPP = 32 * 40 * 256

RC = NCP // 16
RP = NPP // 16
RB = BP // 16

NTILE = 16
_SC_PARAMS = pltpu.CompilerParams(
    use_tc_tiling_on_sc=False, needs_layout_passes=False)


def _mesh():
    return plsc.VectorSubcoreMesh(
        core_axis_name="c", subcore_axis_name="s", num_cores=2, num_subcores=16
    )


def _zero_2d(ref, n_rows, w):
    offs = list(range(0, w - 15, 16))
    if w % 16:
        offs.append(w - 16)
    z16 = jnp.zeros((16,), F32)

    def row(i, carry):
        for j in offs:
            ref[i, pl.ds(j, 16)] = z16
        return carry

    lax.fori_loop(0, n_rows, row, 0)


def _deg_kernel():
    slab_max = ECP // NTILE

    def body(srcc, dstc, srcp, dstp, gidc, gidp, iota_hbm,
             o_sc, o_dc, o_sp, o_dp, o_cc, o_cp,
             tbl_v, slab_v, iota_c, iota_p, iota_b, zer_v,
             sh_a, sh_b, sh_c, sh_d):
        c = lax.axis_index("c")
        s = lax.axis_index("s")
        ones16 = jnp.ones((16,), F32)

        pltpu.sync_copy(iota_hbm.at[pl.ds(0, RC)], iota_c)
        pltpu.sync_copy(iota_hbm.at[pl.ds(0, RP)], iota_p)
        pltpu.sync_copy(iota_hbm.at[pl.ds(0, RB)], iota_b)
        _zero_2d(zer_v, RC // 16, 16)

        def zero_shared(sh, nrows):
            per = nrows // 16
            rem = nrows - per * 16
            if per:
                pltpu.sync_copy(zer_v.at[pl.ds(0, per)],
                                sh.at[pl.ds(s * per, per)])
            if rem:
                @pl.when(s == 0)
                def _():
                    pltpu.sync_copy(zer_v.at[pl.ds(0, rem)],
                                    sh.at[pl.ds(per * 16, rem)])

        def phase(idx_hbm, n_idx_total, nrows, sh, iota_v):
            def zrow(i, carry):
                tbl_v[i, pl.ds(0, 16)] = jnp.zeros((16,), F32)
                return carry
            lax.fori_loop(0, nrows, zrow, 0)
            cnt = n_idx_total // NTILE
            pltpu.sync_copy(idx_hbm.at[pl.ds(s * cnt, cnt)],
                            slab_v.at[pl.ds(0, cnt)])
            def grp(g, carry):
                idx = slab_v[pl.ds(g * 16, 16)]
                r = lax.shift_right_logical(idx, 4)
                q = lax.bitwise_and(idx, 15)
                plsc.addupdate_scatter(tbl_v, (r, q), ones16)
                return carry
            lax.fori_loop(0, cnt // 16, grp, 0)
            pltpu.sync_copy(tbl_v.at[pl.ds(0, nrows)], sh.at[iota_v],
                            add=True)

        def writeout(sh, out, nrows):
            per = nrows // 16
            rem = nrows - per * 16
            pltpu.sync_copy(sh.at[pl.ds(s * per, per)],
                            out.at[pl.ds(s * per, per)])
            if rem:
                @pl.when(s == 0)
                def _():
                    pltpu.sync_copy(sh.at[pl.ds(per * 16, rem)],
                                    out.at[pl.ds(per * 16, rem)])

        @pl.when(c == 0)
        def _():
            zero_shared(sh_a, RC)
            zero_shared(sh_b, RC)
            plsc.subcore_barrier()
            phase(srcc, ECP, RC, sh_a, iota_c)
            phase(dstc, ECP, RC, sh_b, iota_c)
            plsc.subcore_barrier()
            writeout(sh_a, o_sc, RC)
            writeout(sh_b, o_dc, RC)

        @pl.when(c == 1)
        def _():
            zero_shared(sh_a, RP)
            zero_shared(sh_b, RP)
            zero_shared(sh_c, RB)
            zero_shared(sh_d, RB)
            plsc.subcore_barrier()
            phase(srcp, EPP, RP, sh_a, iota_p)
            phase(dstp, EPP, RP, sh_b, iota_p)
            phase(gidc, NCP, RB, sh_c, iota_b)
            phase(gidp, NPP, RB, sh_d, iota_b)
            plsc.subcore_barrier()
            writeout(sh_a, o_sp, RP)
            writeout(sh_b, o_dp, RP)
            writeout(sh_c, o_cc, RB)
            writeout(sh_d, o_cp, RB)

    return pl.kernel(
        body,
        out_type=[
            jax.ShapeDtypeStruct((RC, 16), F32),
            jax.ShapeDtypeStruct((RC, 16), F32),
            jax.ShapeDtypeStruct((RP, 16), F32),
            jax.ShapeDtypeStruct((RP, 16), F32),
            jax.ShapeDtypeStruct((RB, 16), F32),
            jax.ShapeDtypeStruct((RB, 16), F32),
        ],
        mesh=_mesh(),
        compiler_params=_SC_PARAMS,
        scratch_types=[
            pltpu.VMEM((RC, 16), F32),
            pltpu.VMEM((slab_max,), I32),
            pltpu.VMEM((RC,), I32),
            pltpu.VMEM((RP,), I32),
            pltpu.VMEM((RB,), I32),
            pltpu.VMEM((RC // 16, 16), F32),
            pltpu.VMEM_SHARED((RC, 16), F32),
            pltpu.VMEM_SHARED((RC, 16), F32),
            pltpu.VMEM_SHARED((RB, 16), F32),
            pltpu.VMEM_SHARED((RB, 16), F32),
        ],
    )


def _agg_slices(n_pad, w_core, n_slices, e_pad, e_chk, zrows):
    rpt = n_pad // NTILE
    et = e_pad // NTILE
    nchk = et // e_chk
    spc = n_slices // 2
    assert rpt % zrows == 0 and nchk * e_chk == et

    def body(*refs):
        h_refs = refs[:n_slices]
        src_hbm, dst_hbm = refs[n_slices:n_slices + 2]
        out_refs = refs[n_slices + 2:2 * n_slices + 2]
        idxs_v, idxd_v, rows_v, zer_v, tbl_sh = refs[2 * n_slices + 2:]
        c = lax.axis_index("c")
        s = lax.axis_index("s")
        r0 = s * rpt
        e_base = s * et
        _zero_2d(zer_v, zrows, w_core)

        def one_slice(h_hbm, out_hbm):
            for zi in range(rpt // zrows):
                pltpu.sync_copy(zer_v, tbl_sh.at[pl.ds(r0 + zi * zrows, zrows)])
            plsc.subcore_barrier()

            def chunk(i, carry):
                e0 = e_base + i * e_chk
                pltpu.sync_copy(src_hbm.at[pl.ds(e0, e_chk)], idxs_v)
                pltpu.sync_copy(dst_hbm.at[pl.ds(e0, e_chk)], idxd_v)
                pltpu.sync_copy(h_hbm.at[idxs_v], rows_v)
                pltpu.sync_copy(rows_v, tbl_sh.at[idxd_v], add=True)
                return carry

            lax.fori_loop(0, nchk, chunk, 0)
            plsc.subcore_barrier()
            pltpu.sync_copy(tbl_sh.at[pl.ds(r0, rpt)],
                            out_hbm.at[pl.ds(r0, rpt)])

        for ci in range(2):
            @pl.when(c == ci)
            def _(ci=ci):
                for p in range(spc):
                    one_slice(h_refs[ci * spc + p], out_refs[ci * spc + p])

    return pl.kernel(
        body,
        out_type=[jax.ShapeDtypeStruct((n_pad, w_core), F32)
                  for _ in range(n_slices)],
        mesh=_mesh(),
        compiler_params=_SC_PARAMS,
        scratch_types=[
            pltpu.VMEM((e_chk,), I32),
            pltpu.VMEM((e_chk,), I32),
            pltpu.VMEM((e_chk, w_core), F32),
            pltpu.VMEM((zrows, w_core), F32),
            pltpu.VMEM_SHARED((n_pad, w_core), F32),
        ],
    )


def _agg_full(n_pad, e_pad, e_chk, zrows):
    rpt = n_pad // NTILE
    et = e_pad // 32
    nchk = et // e_chk
    assert rpt % zrows == 0 and nchk * e_chk == et

    def body(h_hbm, src_hbm, dst_hbm, out0, out1,
             idxs_v, idxd_v, rows_v, zer_v, tbl_sh):
        c = lax.axis_index("c")
        s = lax.axis_index("s")
        r0 = s * rpt
        e_base = (c * NTILE + s) * et
        _zero_2d(zer_v, zrows, 128)
        for zi in range(rpt // zrows):
            pltpu.sync_copy(zer_v, tbl_sh.at[pl.ds(r0 + zi * zrows, zrows)])
        plsc.subcore_barrier()

        def chunk(i, carry):
            e0 = e_base + i * e_chk
            pltpu.sync_copy(src_hbm.at[pl.ds(e0, e_chk)], idxs_v)
            pltpu.sync_copy(dst_hbm.at[pl.ds(e0, e_chk)], idxd_v)
            pltpu.sync_copy(h_hbm.at[idxs_v], rows_v)
            pltpu.sync_copy(rows_v, tbl_sh.at[idxd_v], add=True)
            return carry

        lax.fori_loop(0, nchk, chunk, 0)
        plsc.subcore_barrier()

        @pl.when(c == 0)
        def _():
            pltpu.sync_copy(tbl_sh.at[pl.ds(r0, rpt)],
                            out0.at[pl.ds(r0, rpt)])

        @pl.when(c == 1)
        def _():
            pltpu.sync_copy(tbl_sh.at[pl.ds(r0, rpt)],
                            out1.at[pl.ds(r0, rpt)])

    return pl.kernel(
        body,
        out_type=[jax.ShapeDtypeStruct((n_pad, 128), F32),
                  jax.ShapeDtypeStruct((n_pad, 128), F32)],
        mesh=_mesh(),
        compiler_params=_SC_PARAMS,
        scratch_types=[
            pltpu.VMEM((e_chk,), I32),
            pltpu.VMEM((e_chk,), I32),
            pltpu.VMEM((e_chk, 128), F32),
            pltpu.VMEM((zrows, 128), F32),
            pltpu.VMEM_SHARED((n_pad, 128), F32),
        ],
    )


def _pool_kernel(n_pad, r_chk):
    rpt = n_pad // 32
    nchk = rpt // r_chk
    brows = BP // NTILE
    assert nchk * r_chk == rpt

    def body(feat_hbm, gid_hbm, out0, out1, gid_v, rows_v, zer_v, tbl_sh):
        c = lax.axis_index("c")
        s = lax.axis_index("s")
        node0 = (c * NTILE + s) * rpt
        _zero_2d(zer_v, brows, 128)
        pltpu.sync_copy(zer_v, tbl_sh.at[pl.ds(s * brows, brows)])
        plsc.subcore_barrier()

        def chunk(i, carry):
            n0 = node0 + i * r_chk
            pltpu.sync_copy(gid_hbm.at[pl.ds(n0, r_chk)], gid_v)
            pltpu.sync_copy(feat_hbm.at[pl.ds(n0, r_chk)], rows_v)
            pltpu.sync_copy(rows_v, tbl_sh.at[gid_v], add=True)
            return carry

        lax.fori_loop(0, nchk, chunk, 0)
        plsc.subcore_barrier()

        @pl.when(c == 0)
        def _():
            pltpu.sync_copy(tbl_sh.at[pl.ds(s * brows, brows)],
                            out0.at[pl.ds(s * brows, brows)])

        @pl.when(c == 1)
        def _():
            pltpu.sync_copy(tbl_sh.at[pl.ds(s * brows, brows)],
                            out1.at[pl.ds(s * brows, brows)])

    return pl.kernel(
        body,
        out_type=[jax.ShapeDtypeStruct((BP, 128), F32),
                  jax.ShapeDtypeStruct((BP, 128), F32)],
        mesh=_mesh(),
        compiler_params=_SC_PARAMS,
        scratch_types=[
            pltpu.VMEM((r_chk,), I32),
            pltpu.VMEM((r_chk, 128), F32),
            pltpu.VMEM((brows, 128), F32),
            pltpu.VMEM_SHARED((BP, 128), F32),
        ],
    )


_BM = 512


def _tc_scale_c(x, dsrc, ddst):
    n, w = x.shape

    def body(x_ref, ds_ref, dd_ref, x0, x1, x2, x3, ro_ref, ri_ref):
        ro = lax.rsqrt(jnp.maximum(ds_ref[...], 1.0))
        ri = lax.rsqrt(jnp.maximum(dd_ref[...], 1.0))
        xp = x_ref[...] * ro
        for i, o in enumerate((x0, x1, x2, x3)):
            o[...] = xp[:, i * 32:(i + 1) * 32]
        ro_ref[...] = ro
        ri_ref[...] = ri

    return pl.pallas_call(
        body,
        grid=(n // _BM,),
        in_specs=[
            pl.BlockSpec((_BM, w), lambda i: (i, 0)),
            pl.BlockSpec((_BM, 1), lambda i: (i, 0)),
            pl.BlockSpec((_BM, 1), lambda i: (i, 0)),
        ],
        out_specs=[pl.BlockSpec((_BM, 32), lambda i: (i, 0))] * 4 + [
            pl.BlockSpec((_BM, 1), lambda i: (i, 0)),
            pl.BlockSpec((_BM, 1), lambda i: (i, 0)),
        ],
        out_shape=[jax.ShapeDtypeStruct((n, 32), F32)] * 4 + [
            jax.ShapeDtypeStruct((n, 1), F32),
            jax.ShapeDtypeStruct((n, 1), F32),
        ],
    )(x, dsrc, ddst)


def _tc_scale_p(x, dsrc, ddst):
    n, w = x.shape

    def body(x_ref, ds_ref, dd_ref, xp_ref, ro_ref, ri_ref):
        ro = lax.rsqrt(jnp.maximum(ds_ref[...], 1.0))
        ri = lax.rsqrt(jnp.maximum(dd_ref[...], 1.0))
        xp_ref[...] = x_ref[...] * ro
        ro_ref[...] = ro
        ri_ref[...] = ri

    return pl.pallas_call(
        body,
        grid=(n // _BM,),
        in_specs=[
            pl.BlockSpec((_BM, w), lambda i: (i, 0)),
            pl.BlockSpec((_BM, 1), lambda i: (i, 0)),
            pl.BlockSpec((_BM, 1), lambda i: (i, 0)),
        ],
        out_specs=[
            pl.BlockSpec((_BM, w), lambda i: (i, 0)),
            pl.BlockSpec((_BM, 1), lambda i: (i, 0)),
            pl.BlockSpec((_BM, 1), lambda i: (i, 0)),
        ],
        out_shape=[
            jax.ShapeDtypeStruct((n, w), F32),
            jax.ShapeDtypeStruct((n, 1), F32),
            jax.ShapeDtypeStruct((n, 1), F32),
        ],
    )(x, dsrc, ddst)


def _tc_mid_c(a_slices, ri, ro, w1, b1, w2):
    n = a_slices[0].shape[0]

    def body(a0, a1, a2, a3, ri_ref, ro_ref, w1_ref, b1_ref, w2_ref,
             o0, o1, o2, o3):
        t = jnp.dot(a0[...], w1_ref[0:32, :], preferred_element_type=F32)
        t += jnp.dot(a1[...], w1_ref[32:64, :], preferred_element_type=F32)
        t += jnp.dot(a2[...], w1_ref[64:96, :], preferred_element_type=F32)
        t += jnp.dot(a3[...], w1_ref[96:128, :], preferred_element_type=F32)
        t = jnp.maximum(t * ri_ref[...] + b1_ref[...], 0.0)
        h2 = jnp.dot(t * ro_ref[...], w2_ref[...], preferred_element_type=F32)
        for i, o in enumerate((o0, o1, o2, o3)):
            o[...] = h2[:, i * 32:(i + 1) * 32]

    return pl.pallas_call(
        body,
        grid=(n // _BM,),
        in_specs=[pl.BlockSpec((_BM, 32), lambda i: (i, 0))] * 4 + [
            pl.BlockSpec((_BM, 1), lambda i: (i, 0)),
            pl.BlockSpec((_BM, 1), lambda i: (i, 0)),
            pl.BlockSpec((HID, HID), lambda i: (0, 0)),
            pl.BlockSpec((1, HID), lambda i: (0, 0)),
            pl.BlockSpec((HID, HID), lambda i: (0, 0)),
        ],
        out_specs=[pl.BlockSpec((_BM, 32), lambda i: (i, 0))] * 4,
        out_shape=[jax.ShapeDtypeStruct((n, 32), F32)] * 4,
    )(*a_slices, ri, ro, w1, b1, w2)


def _tc_mid_p(a0, a1, ri, ro, w1, b1, w2):
    n = a0.shape[0]

    def body(a0_ref, a1_ref, ri_ref, ro_ref, w1_ref, b1_ref, w2_ref, out):
        a = a0_ref[...] + a1_ref[...]
        t = jnp.dot(a, w1_ref[...], preferred_element_type=F32)
        t = jnp.maximum(t * ri_ref[...] + b1_ref[...], 0.0)
        out[...] = jnp.dot(t * ro_ref[...], w2_ref[...],
                           preferred_element_type=F32)

    return pl.pallas_call(
        body,
        grid=(n // _BM,),
        in_specs=[
            pl.BlockSpec((_BM, HID), lambda i: (i, 0)),
            pl.BlockSpec((_BM, HID), lambda i: (i, 0)),
            pl.BlockSpec((_BM, 1), lambda i: (i, 0)),
            pl.BlockSpec((_BM, 1), lambda i: (i, 0)),
            pl.BlockSpec((HID, HID), lambda i: (0, 0)),
            pl.BlockSpec((1, HID), lambda i: (0, 0)),
            pl.BlockSpec((HID, HID), lambda i: (0, 0)),
        ],
        out_specs=pl.BlockSpec((_BM, HID), lambda i: (i, 0)),
        out_shape=jax.ShapeDtypeStruct((n, HID), F32),
    )(a0, a1, ri, ro, w1, b1, w2)


def _tc_post_c(gs, ri, b2):
    n = gs[0].shape[0]

    def body(g0, g1, g2, g3, ri_ref, b_ref, out):
        cf = jnp.concatenate([g0[...], g1[...], g2[...], g3[...]], axis=1)
        out[...] = jnp.maximum(cf * ri_ref[...] + b_ref[...], 0.0)

    return pl.pallas_call(
        body,
        grid=(n // _BM,),
        in_specs=[pl.BlockSpec((_BM, 32), lambda i: (i, 0))] * 4 + [
            pl.BlockSpec((_BM, 1), lambda i: (i, 0)),
            pl.BlockSpec((1, HID), lambda i: (0, 0)),
        ],
        out_specs=pl.BlockSpec((_BM, HID), lambda i: (i, 0)),
        out_shape=jax.ShapeDtypeStruct((n, HID), F32),
    )(*gs, ri, b2)


def _tc_post_p(g0, g1, ri, b2):
    n = g0.shape[0]

    def body(g0_ref, g1_ref, ri_ref, b_ref, out):
        out[...] = jnp.maximum(
            (g0_ref[...] + g1_ref[...]) * ri_ref[...] + b_ref[...], 0.0)

    return pl.pallas_call(
        body,
        grid=(n // _BM,),
        in_specs=[
            pl.BlockSpec((_BM, HID), lambda i: (i, 0)),
            pl.BlockSpec((_BM, HID), lambda i: (i, 0)),
            pl.BlockSpec((_BM, 1), lambda i: (i, 0)),
            pl.BlockSpec((1, HID), lambda i: (0, 0)),
        ],
        out_specs=pl.BlockSpec((_BM, HID), lambda i: (i, 0)),
        out_shape=jax.ShapeDtypeStruct((n, HID), F32),
    )(g0, g1, ri, b2)


def _tc_final(p0, p1, q0, q1, cc, cp, w1a, w1b, b1, w2r, b2):
    def body(p0_ref, p1_ref, q0_ref, q1_ref, cc_ref, cp_ref,
             w1a_ref, w1b_ref, b1_ref, w2_ref, b2_ref, out_ref):
        mc = (p0_ref[...] + p1_ref[...]) / jnp.maximum(cc_ref[...], 1.0)
        mp = (q0_ref[...] + q1_ref[...]) / jnp.maximum(cp_ref[...], 1.0)
        h = jnp.dot(mc, w1a_ref[...], preferred_element_type=F32)
        h += jnp.dot(mp, w1b_ref[...], preferred_element_type=F32)
        h = jnp.maximum(h + b1_ref[...], 0.0)
        out_ref[...] = jnp.sum(h * w2_ref[...], axis=1, keepdims=True) \
            + b2_ref[...]

    return pl.pallas_call(
        body,
        out_shape=jax.ShapeDtypeStruct((NG, 1), F32),
    )(p0, p1, q0, q1, cc, cp, w1a, w1b, b1, w2r, b2)


def kernel(compound_x, compound_edge_index, compound_graph_ids,
           protein_x, protein_edge_index, protein_graph_ids,
           Wc1, bc1, Wc2, bc2, Wp1, bp1, Wp2, bp2, Wm1, bm1, Wm2, bm2):
    dump_c = jnp.full((ECP - EC,), NCP - 8, I32)
    dump_p = jnp.full((EPP - EP,), NPP - 8, I32)

    xc = jnp.pad(compound_x, ((0, NCP - NC), (0, HID - C_IN)))
    src_c = jnp.concatenate([compound_edge_index[0], dump_c])
    dst_c = jnp.concatenate([compound_edge_index[1], dump_c])
    gid_c = jnp.concatenate(
        [compound_graph_ids, jnp.full((NCP - NC,), NG, I32)])

    xp = jnp.pad(protein_x, ((0, NPP - NP2), (0, 0)))
    src_p = jnp.concatenate([protein_edge_index[0], dump_p])
    dst_p = jnp.concatenate([protein_edge_index[1], dump_p])
    gid_p = jnp.concatenate(
        [protein_graph_ids, jnp.full((NPP - NP2,), NG, I32)])

    iota = jnp.arange(RC, dtype=I32)

    dsc, ddc, dsp, ddp, cc, cp = _deg_kernel()(
        src_c, dst_c, src_p, dst_p, gid_c, gid_p, iota)
    dsc = dsc.reshape(-1, 1)
    ddc = ddc.reshape(-1, 1)
    dsp = dsp.reshape(-1, 1)
    ddp = ddp.reshape(-1, 1)
    cc = cc.reshape(-1, 1)[:NG]
    cp = cp.reshape(-1, 1)[:NG]

    w1c = jnp.pad(Wc1, ((0, HID - C_IN), (0, 0)))
    b1c = bc1.reshape(1, HID)
    b2c = bc2.reshape(1, HID)
    b1p = bp1.reshape(1, HID)
    b2p = bp2.reshape(1, HID)

    x0, x1, x2, x3, ro_c, ri_c = _tc_scale_c(xc, dsc, ddc)
    agg_c1 = _agg_slices(NCP, 32, 4, ECP, 512, 112)
    a0, a1, a2, a3 = agg_c1(x0, x1, x2, x3, src_c, dst_c)
    h0, h1, h2, h3 = _tc_mid_c((a0, a1, a2, a3), ri_c, ro_c, w1c, b1c, Wc2)
    agg_c2 = _agg_slices(NCP, 32, 4, ECP, 512, 112)
    g0, g1, g2, g3 = agg_c2(h0, h1, h2, h3, src_c, dst_c)
    cf = _tc_post_c((g0, g1, g2, g3), ri_c, b2c)
    p0, p1 = _pool_kernel(NCP, 224)(cf, gid_c)

    yp, ro_p, ri_p = _tc_scale_p(xp, dsp, ddp)
    b0, b1_ = _agg_full(NPP, EPP, 256, 40)(yp, src_p, dst_p)
    h2p = _tc_mid_p(b0, b1_, ri_p, ro_p, Wp1, b1p, Wp2)
    gp0, gp1 = _agg_full(NPP, EPP, 256, 40)(h2p, src_p, dst_p)
    pf = _tc_post_p(gp0, gp1, ri_p, b2p)
    q0, q1 = _pool_kernel(NPP, 320)(pf, gid_p)

    out = _tc_final(p0[:NG], p1[:NG], q0[:NG], q1[:NG], cc, cp,
                    Wm1[:HID], Wm1[HID:], bm1.reshape(1, HID),
                    Wm2.reshape(1, HID), bm2.reshape(1, 1))
    return jnp.squeeze(out, axis=-1)


# speedup vs baseline: 3.7091x; 1.1758x over previous
import jax
import jax.numpy as jnp
from jax import lax
from jax.experimental import pallas as pl
from jax.experimental.pallas import tpu as pltpu
from jax.experimental.pallas import tpu_sc as plsc

F32 = jnp.float32
I32 = jnp.int32

NC, EC = 50000, 800000
NP2, EP = 10000, 320000
NG = 1024
C_IN, P_IN, HID = 74, 128, 128

NCP = 50176
NPP = 10240
BP = 1040
ECP = 16 * 98 * 512
EPP = 32 * 40 * 256

RC = NCP // 16
RP = NPP // 16
RB = BP // 16

NTILE = 16
_SC_PARAMS = pltpu.CompilerParams(
    use_tc_tiling_on_sc=False, needs_layout_passes=False)


def _mesh():
    return plsc.VectorSubcoreMesh(
        core_axis_name="c", subcore_axis_name="s", num_cores=2, num_subcores=16
    )


def _zero_2d(ref, n_rows, w):
    offs = list(range(0, w - 15, 16))
    if w % 16:
        offs.append(w - 16)
    z16 = jnp.zeros((16,), F32)

    def row(i, carry):
        for j in offs:
            ref[i, pl.ds(j, 16)] = z16
        return carry

    lax.fori_loop(0, n_rows, row, 0)


def _deg_kernel():
    slab_max = ECP // NTILE

    def body(srcc, dstc, srcp, dstp, gidc, gidp, iota_hbm,
             o_sc, o_dc, o_sp, o_dp, o_cc, o_cp,
             tbl_v, slab_v, iota_c, iota_p, iota_b, zer_v,
             sh_a, sh_b, sh_c, sh_d):
        c = lax.axis_index("c")
        s = lax.axis_index("s")
        ones16 = jnp.ones((16,), F32)

        pltpu.sync_copy(iota_hbm.at[pl.ds(0, RC)], iota_c)
        pltpu.sync_copy(iota_hbm.at[pl.ds(0, RP)], iota_p)
        pltpu.sync_copy(iota_hbm.at[pl.ds(0, RB)], iota_b)
        _zero_2d(zer_v, RC // 16, 16)

        def zero_shared(sh, nrows):
            per = nrows // 16
            rem = nrows - per * 16
            if per:
                pltpu.sync_copy(zer_v.at[pl.ds(0, per)],
                                sh.at[pl.ds(s * per, per)])
            if rem:
                @pl.when(s == 0)
                def _():
                    pltpu.sync_copy(zer_v.at[pl.ds(0, rem)],
                                    sh.at[pl.ds(per * 16, rem)])

        def phase(idx_hbm, n_idx_total, nrows, sh, iota_v):
            def zrow(i, carry):
                tbl_v[i, pl.ds(0, 16)] = jnp.zeros((16,), F32)
                return carry
            lax.fori_loop(0, nrows, zrow, 0)
            cnt = n_idx_total // NTILE
            pltpu.sync_copy(idx_hbm.at[pl.ds(s * cnt, cnt)],
                            slab_v.at[pl.ds(0, cnt)])
            def grp(g, carry):
                idx = slab_v[pl.ds(g * 16, 16)]
                r = lax.shift_right_logical(idx, 4)
                q = lax.bitwise_and(idx, 15)
                plsc.addupdate_scatter(tbl_v, (r, q), ones16)
                return carry
            lax.fori_loop(0, cnt // 16, grp, 0)
            pltpu.sync_copy(tbl_v.at[pl.ds(0, nrows)], sh.at[iota_v],
                            add=True)

        def writeout(sh, out, nrows):
            per = nrows // 16
            rem = nrows - per * 16
            pltpu.sync_copy(sh.at[pl.ds(s * per, per)],
                            out.at[pl.ds(s * per, per)])
            if rem:
                @pl.when(s == 0)
                def _():
                    pltpu.sync_copy(sh.at[pl.ds(per * 16, rem)],
                                    out.at[pl.ds(per * 16, rem)])

        @pl.when(c == 0)
        def _():
            zero_shared(sh_a, RC)
            zero_shared(sh_b, RC)
            plsc.subcore_barrier()
            phase(srcc, ECP, RC, sh_a, iota_c)
            phase(dstc, ECP, RC, sh_b, iota_c)
            plsc.subcore_barrier()
            writeout(sh_a, o_sc, RC)
            writeout(sh_b, o_dc, RC)

        @pl.when(c == 1)
        def _():
            zero_shared(sh_a, RP)
            zero_shared(sh_b, RP)
            zero_shared(sh_c, RB)
            zero_shared(sh_d, RB)
            plsc.subcore_barrier()
            phase(srcp, EPP, RP, sh_a, iota_p)
            phase(dstp, EPP, RP, sh_b, iota_p)
            phase(gidc, NCP, RB, sh_c, iota_b)
            phase(gidp, NPP, RB, sh_d, iota_b)
            plsc.subcore_barrier()
            writeout(sh_a, o_sp, RP)
            writeout(sh_b, o_dp, RP)
            writeout(sh_c, o_cc, RB)
            writeout(sh_d, o_cp, RB)

    return pl.kernel(
        body,
        out_type=[
            jax.ShapeDtypeStruct((RC, 16), F32),
            jax.ShapeDtypeStruct((RC, 16), F32),
            jax.ShapeDtypeStruct((RP, 16), F32),
            jax.ShapeDtypeStruct((RP, 16), F32),
            jax.ShapeDtypeStruct((RB, 16), F32),
            jax.ShapeDtypeStruct((RB, 16), F32),
        ],
        mesh=_mesh(),
        compiler_params=_SC_PARAMS,
        scratch_types=[
            pltpu.VMEM((RC, 16), F32),
            pltpu.VMEM((slab_max,), I32),
            pltpu.VMEM((RC,), I32),
            pltpu.VMEM((RP,), I32),
            pltpu.VMEM((RB,), I32),
            pltpu.VMEM((RC // 16, 16), F32),
            pltpu.VMEM_SHARED((RC, 16), F32),
            pltpu.VMEM_SHARED((RC, 16), F32),
            pltpu.VMEM_SHARED((RB, 16), F32),
            pltpu.VMEM_SHARED((RB, 16), F32),
        ],
    )


def _agg_slices(n_pad, w_core, n_slices, e_pad, e_chk, grp, zrows):
    rpt = n_pad // NTILE
    rows_per_tile = e_pad // e_chk // NTILE
    ngrp = rows_per_tile // grp
    spc = n_slices // 2
    assert rpt % zrows == 0 and ngrp * grp == rows_per_tile

    def body(*refs):
        h_refs = refs[:n_slices]
        src_hbm, dst_hbm = refs[n_slices:n_slices + 2]
        out_refs = refs[n_slices + 2:2 * n_slices + 2]
        (idxs_v, idxd_v, rows_v, zer_v, tbl_sh, sem0, sem1) = \
            refs[2 * n_slices + 2:]
        sems = (sem0, sem1)
        c = lax.axis_index("c")
        s = lax.axis_index("s")
        r0 = s * rpt
        row_base = s * rows_per_tile
        _zero_2d(zer_v, zrows, w_core)

        def one_slice(h_hbm, out_hbm):
            for zi in range(rpt // zrows):
                pltpu.sync_copy(zer_v, tbl_sh.at[pl.ds(r0 + zi * zrows, zrows)])
            plsc.subcore_barrier()

            def group(gi, carry):
                c0 = row_base + gi * grp
                pltpu.sync_copy(src_hbm.at[pl.ds(c0, grp)], idxs_v)
                pltpu.sync_copy(dst_hbm.at[pl.ds(c0, grp)], idxd_v)
                cps = [pltpu.async_copy(h_hbm.at[idxs_v.at[0]],
                                        rows_v.at[0], sems[0])]
                for j in range(grp):
                    if j + 1 < grp:
                        cps.append(pltpu.async_copy(
                            h_hbm.at[idxs_v.at[j + 1]],
                            rows_v.at[(j + 1) % 2], sems[(j + 1) % 2]))
                    cps[j].wait()
                    pltpu.sync_copy(rows_v.at[j % 2],
                                    tbl_sh.at[idxd_v.at[j]], add=True)
                return carry

            lax.fori_loop(0, ngrp, group, 0)
            plsc.subcore_barrier()
            pltpu.sync_copy(tbl_sh.at[pl.ds(r0, rpt)],
                            out_hbm.at[pl.ds(r0, rpt)])

        for ci in range(2):
            @pl.when(c == ci)
            def _(ci=ci):
                for p in range(spc):
                    one_slice(h_refs[ci * spc + p], out_refs[ci * spc + p])

    return pl.kernel(
        body,
        out_type=[jax.ShapeDtypeStruct((n_pad, w_core), F32)
                  for _ in range(n_slices)],
        mesh=_mesh(),
        compiler_params=_SC_PARAMS,
        scratch_types=[
            pltpu.VMEM((grp, e_chk), I32),
            pltpu.VMEM((grp, e_chk), I32),
            pltpu.VMEM((2, e_chk, w_core), F32),
            pltpu.VMEM((zrows, w_core), F32),
            pltpu.VMEM_SHARED((n_pad, w_core), F32),
            pltpu.SemaphoreType.DMA,
            pltpu.SemaphoreType.DMA,
        ],
    )


def _agg_full(n_pad, e_pad, e_chk, grp, zrows):
    rpt = n_pad // NTILE
    rows_per_w = e_pad // e_chk // 32
    ngrp = rows_per_w // grp
    assert rpt % zrows == 0 and ngrp * grp == rows_per_w

    def body(h_hbm, src_hbm, dst_hbm, out0, out1,
             idxs_v, idxd_v, rows_v, zer_v, tbl_sh, sem0, sem1):
        sems = (sem0, sem1)
        c = lax.axis_index("c")
        s = lax.axis_index("s")
        r0 = s * rpt
        row_base = (c * NTILE + s) * rows_per_w
        _zero_2d(zer_v, zrows, 128)
        for zi in range(rpt // zrows):
            pltpu.sync_copy(zer_v, tbl_sh.at[pl.ds(r0 + zi * zrows, zrows)])
        plsc.subcore_barrier()

        def group(gi, carry):
            c0 = row_base + gi * grp
            pltpu.sync_copy(src_hbm.at[pl.ds(c0, grp)], idxs_v)
            pltpu.sync_copy(dst_hbm.at[pl.ds(c0, grp)], idxd_v)
            cps = [pltpu.async_copy(h_hbm.at[idxs_v.at[0]],
                                    rows_v.at[0], sems[0])]
            for j in range(grp):
                if j + 1 < grp:
                    cps.append(pltpu.async_copy(
                        h_hbm.at[idxs_v.at[j + 1]],
                        rows_v.at[(j + 1) % 2], sems[(j + 1) % 2]))
                cps[j].wait()
                pltpu.sync_copy(rows_v.at[j % 2],
                                tbl_sh.at[idxd_v.at[j]], add=True)
            return carry

        lax.fori_loop(0, ngrp, group, 0)
        plsc.subcore_barrier()

        @pl.when(c == 0)
        def _():
            pltpu.sync_copy(tbl_sh.at[pl.ds(r0, rpt)],
                            out0.at[pl.ds(r0, rpt)])

        @pl.when(c == 1)
        def _():
            pltpu.sync_copy(tbl_sh.at[pl.ds(r0, rpt)],
                            out1.at[pl.ds(r0, rpt)])

    return pl.kernel(
        body,
        out_type=[jax.ShapeDtypeStruct((n_pad, 128), F32),
                  jax.ShapeDtypeStruct((n_pad, 128), F32)],
        mesh=_mesh(),
        compiler_params=_SC_PARAMS,
        scratch_types=[
            pltpu.VMEM((grp, e_chk), I32),
            pltpu.VMEM((grp, e_chk), I32),
            pltpu.VMEM((2, e_chk, 128), F32),
            pltpu.VMEM((zrows, 128), F32),
            pltpu.VMEM_SHARED((n_pad, 128), F32),
            pltpu.SemaphoreType.DMA,
            pltpu.SemaphoreType.DMA,
        ],
    )


def _pool_kernel(n_pad, r_chk):
    rpt = n_pad // 32
    nchk = rpt // r_chk
    brows = BP // NTILE
    assert nchk * r_chk == rpt

    def body(feat_hbm, gid_hbm, out0, out1, gid_v, rows_v, zer_v, tbl_sh):
        c = lax.axis_index("c")
        s = lax.axis_index("s")
        node0 = (c * NTILE + s) * rpt
        _zero_2d(zer_v, brows, 128)
        pltpu.sync_copy(zer_v, tbl_sh.at[pl.ds(s * brows, brows)])
        plsc.subcore_barrier()

        def chunk(i, carry):
            n0 = node0 + i * r_chk
            pltpu.sync_copy(gid_hbm.at[pl.ds(n0, r_chk)], gid_v)
            pltpu.sync_copy(feat_hbm.at[pl.ds(n0, r_chk)], rows_v)
            pltpu.sync_copy(rows_v, tbl_sh.at[gid_v], add=True)
            return carry

        lax.fori_loop(0, nchk, chunk, 0)
        plsc.subcore_barrier()

        @pl.when(c == 0)
        def _():
            pltpu.sync_copy(tbl_sh.at[pl.ds(s * brows, brows)],
                            out0.at[pl.ds(s * brows, brows)])

        @pl.when(c == 1)
        def _():
            pltpu.sync_copy(tbl_sh.at[pl.ds(s * brows, brows)],
                            out1.at[pl.ds(s * brows, brows)])

    return pl.kernel(
        body,
        out_type=[jax.ShapeDtypeStruct((BP, 128), F32),
                  jax.ShapeDtypeStruct((BP, 128), F32)],
        mesh=_mesh(),
        compiler_params=_SC_PARAMS,
        scratch_types=[
            pltpu.VMEM((r_chk,), I32),
            pltpu.VMEM((r_chk, 128), F32),
            pltpu.VMEM((brows, 128), F32),
            pltpu.VMEM_SHARED((BP, 128), F32),
        ],
    )


_BM = 512


def _tc_scale_c(x, dsrc, ddst):
    n, w = x.shape

    def body(x_ref, ds_ref, dd_ref, x0, x1, x2, x3, ro_ref, ri_ref):
        ro = lax.rsqrt(jnp.maximum(ds_ref[...], 1.0))
        ri = lax.rsqrt(jnp.maximum(dd_ref[...], 1.0))
        xp = x_ref[...] * ro
        for i, o in enumerate((x0, x1, x2, x3)):
            o[...] = xp[:, i * 32:(i + 1) * 32]
        ro_ref[...] = ro
        ri_ref[...] = ri

    return pl.pallas_call(
        body,
        grid=(n // _BM,),
        in_specs=[
            pl.BlockSpec((_BM, w), lambda i: (i, 0)),
            pl.BlockSpec((_BM, 1), lambda i: (i, 0)),
            pl.BlockSpec((_BM, 1), lambda i: (i, 0)),
        ],
        out_specs=[pl.BlockSpec((_BM, 32), lambda i: (i, 0))] * 4 + [
            pl.BlockSpec((_BM, 1), lambda i: (i, 0)),
            pl.BlockSpec((_BM, 1), lambda i: (i, 0)),
        ],
        out_shape=[jax.ShapeDtypeStruct((n, 32), F32)] * 4 + [
            jax.ShapeDtypeStruct((n, 1), F32),
            jax.ShapeDtypeStruct((n, 1), F32),
        ],
    )(x, dsrc, ddst)


def _tc_scale_p(x, dsrc, ddst):
    n, w = x.shape

    def body(x_ref, ds_ref, dd_ref, xp_ref, ro_ref, ri_ref):
        ro = lax.rsqrt(jnp.maximum(ds_ref[...], 1.0))
        ri = lax.rsqrt(jnp.maximum(dd_ref[...], 1.0))
        xp_ref[...] = x_ref[...] * ro
        ro_ref[...] = ro
        ri_ref[...] = ri

    return pl.pallas_call(
        body,
        grid=(n // _BM,),
        in_specs=[
            pl.BlockSpec((_BM, w), lambda i: (i, 0)),
            pl.BlockSpec((_BM, 1), lambda i: (i, 0)),
            pl.BlockSpec((_BM, 1), lambda i: (i, 0)),
        ],
        out_specs=[
            pl.BlockSpec((_BM, w), lambda i: (i, 0)),
            pl.BlockSpec((_BM, 1), lambda i: (i, 0)),
            pl.BlockSpec((_BM, 1), lambda i: (i, 0)),
        ],
        out_shape=[
            jax.ShapeDtypeStruct((n, w), F32),
            jax.ShapeDtypeStruct((n, 1), F32),
            jax.ShapeDtypeStruct((n, 1), F32),
        ],
    )(x, dsrc, ddst)


def _tc_mid_c(a_slices, ri, ro, w1, b1, w2):
    n = a_slices[0].shape[0]

    def body(a0, a1, a2, a3, ri_ref, ro_ref, w1_ref, b1_ref, w2_ref,
             o0, o1, o2, o3):
        t = jnp.dot(a0[...], w1_ref[0:32, :], preferred_element_type=F32)
        t += jnp.dot(a1[...], w1_ref[32:64, :], preferred_element_type=F32)
        t += jnp.dot(a2[...], w1_ref[64:96, :], preferred_element_type=F32)
        t += jnp.dot(a3[...], w1_ref[96:128, :], preferred_element_type=F32)
        t = jnp.maximum(t * ri_ref[...] + b1_ref[...], 0.0)
        h2 = jnp.dot(t * ro_ref[...], w2_ref[...], preferred_element_type=F32)
        for i, o in enumerate((o0, o1, o2, o3)):
            o[...] = h2[:, i * 32:(i + 1) * 32]

    return pl.pallas_call(
        body,
        grid=(n // _BM,),
        in_specs=[pl.BlockSpec((_BM, 32), lambda i: (i, 0))] * 4 + [
            pl.BlockSpec((_BM, 1), lambda i: (i, 0)),
            pl.BlockSpec((_BM, 1), lambda i: (i, 0)),
            pl.BlockSpec((HID, HID), lambda i: (0, 0)),
            pl.BlockSpec((1, HID), lambda i: (0, 0)),
            pl.BlockSpec((HID, HID), lambda i: (0, 0)),
        ],
        out_specs=[pl.BlockSpec((_BM, 32), lambda i: (i, 0))] * 4,
        out_shape=[jax.ShapeDtypeStruct((n, 32), F32)] * 4,
    )(*a_slices, ri, ro, w1, b1, w2)


def _tc_mid_p(a0, a1, ri, ro, w1, b1, w2):
    n = a0.shape[0]

    def body(a0_ref, a1_ref, ri_ref, ro_ref, w1_ref, b1_ref, w2_ref, out):
        a = a0_ref[...] + a1_ref[...]
        t = jnp.dot(a, w1_ref[...], preferred_element_type=F32)
        t = jnp.maximum(t * ri_ref[...] + b1_ref[...], 0.0)
        out[...] = jnp.dot(t * ro_ref[...], w2_ref[...],
                           preferred_element_type=F32)

    return pl.pallas_call(
        body,
        grid=(n // _BM,),
        in_specs=[
            pl.BlockSpec((_BM, HID), lambda i: (i, 0)),
            pl.BlockSpec((_BM, HID), lambda i: (i, 0)),
            pl.BlockSpec((_BM, 1), lambda i: (i, 0)),
            pl.BlockSpec((_BM, 1), lambda i: (i, 0)),
            pl.BlockSpec((HID, HID), lambda i: (0, 0)),
            pl.BlockSpec((1, HID), lambda i: (0, 0)),
            pl.BlockSpec((HID, HID), lambda i: (0, 0)),
        ],
        out_specs=pl.BlockSpec((_BM, HID), lambda i: (i, 0)),
        out_shape=jax.ShapeDtypeStruct((n, HID), F32),
    )(a0, a1, ri, ro, w1, b1, w2)


def _tc_post_c(gs, ri, b2):
    n = gs[0].shape[0]

    def body(g0, g1, g2, g3, ri_ref, b_ref, out):
        cf = jnp.concatenate([g0[...], g1[...], g2[...], g3[...]], axis=1)
        out[...] = jnp.maximum(cf * ri_ref[...] + b_ref[...], 0.0)

    return pl.pallas_call(
        body,
        grid=(n // _BM,),
        in_specs=[pl.BlockSpec((_BM, 32), lambda i: (i, 0))] * 4 + [
            pl.BlockSpec((_BM, 1), lambda i: (i, 0)),
            pl.BlockSpec((1, HID), lambda i: (0, 0)),
        ],
        out_specs=pl.BlockSpec((_BM, HID), lambda i: (i, 0)),
        out_shape=jax.ShapeDtypeStruct((n, HID), F32),
    )(*gs, ri, b2)


def _tc_post_p(g0, g1, ri, b2):
    n = g0.shape[0]

    def body(g0_ref, g1_ref, ri_ref, b_ref, out):
        out[...] = jnp.maximum(
            (g0_ref[...] + g1_ref[...]) * ri_ref[...] + b_ref[...], 0.0)

    return pl.pallas_call(
        body,
        grid=(n // _BM,),
        in_specs=[
            pl.BlockSpec((_BM, HID), lambda i: (i, 0)),
            pl.BlockSpec((_BM, HID), lambda i: (i, 0)),
            pl.BlockSpec((_BM, 1), lambda i: (i, 0)),
            pl.BlockSpec((1, HID), lambda i: (0, 0)),
        ],
        out_specs=pl.BlockSpec((_BM, HID), lambda i: (i, 0)),
        out_shape=jax.ShapeDtypeStruct((n, HID), F32),
    )(g0, g1, ri, b2)


def _tc_final(p0, p1, q0, q1, cc, cp, w1a, w1b, b1, w2r, b2):
    def body(p0_ref, p1_ref, q0_ref, q1_ref, cc_ref, cp_ref,
             w1a_ref, w1b_ref, b1_ref, w2_ref, b2_ref, out_ref):
        mc = (p0_ref[...] + p1_ref[...]) / jnp.maximum(cc_ref[...], 1.0)
        mp = (q0_ref[...] + q1_ref[...]) / jnp.maximum(cp_ref[...], 1.0)
        h = jnp.dot(mc, w1a_ref[...], preferred_element_type=F32)
        h += jnp.dot(mp, w1b_ref[...], preferred_element_type=F32)
        h = jnp.maximum(h + b1_ref[...], 0.0)
        out_ref[...] = jnp.sum(h * w2_ref[...], axis=1, keepdims=True) \
            + b2_ref[...]

    return pl.pallas_call(
        body,
        out_shape=jax.ShapeDtypeStruct((NG, 1), F32),
    )(p0, p1, q0, q1, cc, cp, w1a, w1b, b1, w2r, b2)


def kernel(compound_x, compound_edge_index, compound_graph_ids,
           protein_x, protein_edge_index, protein_graph_ids,
           Wc1, bc1, Wc2, bc2, Wp1, bp1, Wp2, bp2, Wm1, bm1, Wm2, bm2):
    dump_c = jnp.full((ECP - EC,), NCP - 8, I32)
    dump_p = jnp.full((EPP - EP,), NPP - 8, I32)

    xc = jnp.pad(compound_x, ((0, NCP - NC), (0, HID - C_IN)))
    src_c = jnp.concatenate([compound_edge_index[0], dump_c])
    dst_c = jnp.concatenate([compound_edge_index[1], dump_c])
    gid_c = jnp.concatenate(
        [compound_graph_ids, jnp.full((NCP - NC,), NG, I32)])

    xp = jnp.pad(protein_x, ((0, NPP - NP2), (0, 0)))
    src_p = jnp.concatenate([protein_edge_index[0], dump_p])
    dst_p = jnp.concatenate([protein_edge_index[1], dump_p])
    gid_p = jnp.concatenate(
        [protein_graph_ids, jnp.full((NPP - NP2,), NG, I32)])

    iota = jnp.arange(RC, dtype=I32)

    src_c2 = src_c.reshape(-1, 392)
    dst_c2 = dst_c.reshape(-1, 392)
    src_p2 = src_p.reshape(-1, 160)
    dst_p2 = dst_p.reshape(-1, 160)

    dsc, ddc, dsp, ddp, cc, cp = _deg_kernel()(
        src_c, dst_c, src_p, dst_p, gid_c, gid_p, iota)
    dsc = dsc.reshape(-1, 1)
    ddc = ddc.reshape(-1, 1)
    dsp = dsp.reshape(-1, 1)
    ddp = ddp.reshape(-1, 1)
    cc = cc.reshape(-1, 1)[:NG]
    cp = cp.reshape(-1, 1)[:NG]

    w1c = jnp.pad(Wc1, ((0, HID - C_IN), (0, 0)))
    b1c = bc1.reshape(1, HID)
    b2c = bc2.reshape(1, HID)
    b1p = bp1.reshape(1, HID)
    b2p = bp2.reshape(1, HID)

    x0, x1, x2, x3, ro_c, ri_c = _tc_scale_c(xc, dsc, ddc)
    agg_c1 = _agg_slices(NCP, 32, 4, ECP, 392, 4, 28)
    a0, a1, a2, a3 = agg_c1(x0, x1, x2, x3, src_c2, dst_c2)
    h0, h1, h2, h3 = _tc_mid_c((a0, a1, a2, a3), ri_c, ro_c, w1c, b1c, Wc2)
    agg_c2 = _agg_slices(NCP, 32, 4, ECP, 392, 4, 28)
    g0, g1, g2, g3 = agg_c2(h0, h1, h2, h3, src_c2, dst_c2)
    cf = _tc_post_c((g0, g1, g2, g3), ri_c, b2c)
    p0, p1 = _pool_kernel(NCP, 224)(cf, gid_c)

    yp, ro_p, ri_p = _tc_scale_p(xp, dsp, ddp)
    b0, b1_ = _agg_full(NPP, EPP, 160, 4, 20)(yp, src_p2, dst_p2)
    h2p = _tc_mid_p(b0, b1_, ri_p, ro_p, Wp1, b1p, Wp2)
    gp0, gp1 = _agg_full(NPP, EPP, 160, 4, 20)(h2p, src_p2, dst_p2)
    pf = _tc_post_p(gp0, gp1, ri_p, b2p)
    q0, q1 = _pool_kernel(NPP, 320)(pf, gid_p)

    out = _tc_final(p0[:NG], p1[:NG], q0[:NG], q1[:NG], cc, cp,
                    Wm1[:HID], Wm1[HID:], bm1.reshape(1, HID),
                    Wm2.reshape(1, HID), bm2.reshape(1, 1))
    return jnp.squeeze(out, axis=-1)


# speedup vs baseline: 3.7109x; 1.0005x over previous
import jax
import jax.numpy as jnp
from jax import lax
from jax.experimental import pallas as pl
from jax.experimental.pallas import tpu as pltpu
from jax.experimental.pallas import tpu_sc as plsc

F32 = jnp.float32
I32 = jnp.int32

NC, EC = 50000, 800000
NP2, EP = 10000, 320000
NG = 1024
C_IN, P_IN, HID = 74, 128, 128

NCP = 50176
NPP = 10240
BP = 1040
ECP = 16 * 98 * 512
EPP = 32 * 40 * 256

RC = NCP // 16
RP = NPP // 16
RB = BP // 16

NTILE = 16
_SC_PARAMS = pltpu.CompilerParams(
    use_tc_tiling_on_sc=False, needs_layout_passes=False)


def _mesh():
    return plsc.VectorSubcoreMesh(
        core_axis_name="c", subcore_axis_name="s", num_cores=2, num_subcores=16
    )


def _zero_2d(ref, n_rows, w):
    offs = list(range(0, w - 15, 16))
    if w % 16:
        offs.append(w - 16)
    z16 = jnp.zeros((16,), F32)

    def row(i, carry):
        for j in offs:
            ref[i, pl.ds(j, 16)] = z16
        return carry

    lax.fori_loop(0, n_rows, row, 0)


def _deg_kernel():
    slab_max = ECP // NTILE

    def body(srcc, dstc, srcp, dstp, gidc, gidp, iota_hbm,
             o_sc, o_dc, o_sp, o_dp, o_cc, o_cp,
             tbl_v, slab_v, iota_c, iota_p, iota_b, zer_v,
             sh_a, sh_b, sh_c, sh_d):
        c = lax.axis_index("c")
        s = lax.axis_index("s")
        ones16 = jnp.ones((16,), F32)

        pltpu.sync_copy(iota_hbm.at[pl.ds(0, RC)], iota_c)
        pltpu.sync_copy(iota_hbm.at[pl.ds(0, RP)], iota_p)
        pltpu.sync_copy(iota_hbm.at[pl.ds(0, RB)], iota_b)
        _zero_2d(zer_v, RC // 16, 16)

        def zero_shared(sh, nrows):
            per = nrows // 16
            rem = nrows - per * 16
            if per:
                pltpu.sync_copy(zer_v.at[pl.ds(0, per)],
                                sh.at[pl.ds(s * per, per)])
            if rem:
                @pl.when(s == 0)
                def _():
                    pltpu.sync_copy(zer_v.at[pl.ds(0, rem)],
                                    sh.at[pl.ds(per * 16, rem)])

        def phase(idx_hbm, n_idx_total, nrows, sh, iota_v):
            def zrow(i, carry):
                tbl_v[i, pl.ds(0, 16)] = jnp.zeros((16,), F32)
                return carry
            lax.fori_loop(0, nrows, zrow, 0)
            cnt = n_idx_total // NTILE
            pltpu.sync_copy(idx_hbm.at[pl.ds(s * cnt, cnt)],
                            slab_v.at[pl.ds(0, cnt)])
            def grp(g, carry):
                idx = slab_v[pl.ds(g * 16, 16)]
                r = lax.shift_right_logical(idx, 4)
                q = lax.bitwise_and(idx, 15)
                plsc.addupdate_scatter(tbl_v, (r, q), ones16)
                return carry
            lax.fori_loop(0, cnt // 16, grp, 0)
            pltpu.sync_copy(tbl_v.at[pl.ds(0, nrows)], sh.at[iota_v],
                            add=True)

        def writeout(sh, out, nrows):
            per = nrows // 16
            rem = nrows - per * 16
            pltpu.sync_copy(sh.at[pl.ds(s * per, per)],
                            out.at[pl.ds(s * per, per)])
            if rem:
                @pl.when(s == 0)
                def _():
                    pltpu.sync_copy(sh.at[pl.ds(per * 16, rem)],
                                    out.at[pl.ds(per * 16, rem)])

        @pl.when(c == 0)
        def _():
            zero_shared(sh_a, RC)
            zero_shared(sh_b, RC)
            plsc.subcore_barrier()
            phase(srcc, ECP, RC, sh_a, iota_c)
            phase(dstc, ECP, RC, sh_b, iota_c)
            plsc.subcore_barrier()
            writeout(sh_a, o_sc, RC)
            writeout(sh_b, o_dc, RC)

        @pl.when(c == 1)
        def _():
            zero_shared(sh_a, RP)
            zero_shared(sh_b, RP)
            zero_shared(sh_c, RB)
            zero_shared(sh_d, RB)
            plsc.subcore_barrier()
            phase(srcp, EPP, RP, sh_a, iota_p)
            phase(dstp, EPP, RP, sh_b, iota_p)
            phase(gidc, NCP, RB, sh_c, iota_b)
            phase(gidp, NPP, RB, sh_d, iota_b)
            plsc.subcore_barrier()
            writeout(sh_a, o_sp, RP)
            writeout(sh_b, o_dp, RP)
            writeout(sh_c, o_cc, RB)
            writeout(sh_d, o_cp, RB)

    return pl.kernel(
        body,
        out_type=[
            jax.ShapeDtypeStruct((RC, 16), F32),
            jax.ShapeDtypeStruct((RC, 16), F32),
            jax.ShapeDtypeStruct((RP, 16), F32),
            jax.ShapeDtypeStruct((RP, 16), F32),
            jax.ShapeDtypeStruct((RB, 16), F32),
            jax.ShapeDtypeStruct((RB, 16), F32),
        ],
        mesh=_mesh(),
        compiler_params=_SC_PARAMS,
        scratch_types=[
            pltpu.VMEM((RC, 16), F32),
            pltpu.VMEM((slab_max,), I32),
            pltpu.VMEM((RC,), I32),
            pltpu.VMEM((RP,), I32),
            pltpu.VMEM((RB,), I32),
            pltpu.VMEM((RC // 16, 16), F32),
            pltpu.VMEM_SHARED((RC, 16), F32),
            pltpu.VMEM_SHARED((RC, 16), F32),
            pltpu.VMEM_SHARED((RB, 16), F32),
            pltpu.VMEM_SHARED((RB, 16), F32),
        ],
    )


def _agg_slices(n_pad, w_core, n_slices, e_pad, e_chk, grp, zrows):
    rpt = n_pad // NTILE
    rows_per_tile = e_pad // e_chk // NTILE
    ngrp = rows_per_tile // grp
    spc = n_slices // 2
    assert rpt % zrows == 0 and ngrp * grp == rows_per_tile

    def body(*refs):
        h_refs = refs[:n_slices]
        src_hbm, dst_hbm = refs[n_slices:n_slices + 2]
        out_refs = refs[n_slices + 2:2 * n_slices + 2]
        (idxs_v, idxd_v, rows_v, zer_v, tbl_sh, sem0, sem1) = \
            refs[2 * n_slices + 2:]
        sems = (sem0, sem1)
        c = lax.axis_index("c")
        s = lax.axis_index("s")
        r0 = s * rpt
        row_base = s * rows_per_tile
        _zero_2d(zer_v, zrows, w_core)

        def one_slice(h_hbm, out_hbm):
            for zi in range(rpt // zrows):
                pltpu.sync_copy(zer_v, tbl_sh.at[pl.ds(r0 + zi * zrows, zrows)])
            plsc.subcore_barrier()

            def group(gi, carry):
                c0 = row_base + gi * grp
                pltpu.sync_copy(src_hbm.at[pl.ds(c0, grp)], idxs_v)
                pltpu.sync_copy(dst_hbm.at[pl.ds(c0, grp)], idxd_v)
                cps = [pltpu.async_copy(h_hbm.at[idxs_v.at[0]],
                                        rows_v.at[0], sems[0])]
                for j in range(grp):
                    if j + 1 < grp:
                        cps.append(pltpu.async_copy(
                            h_hbm.at[idxs_v.at[j + 1]],
                            rows_v.at[(j + 1) % 2], sems[(j + 1) % 2]))
                    cps[j].wait()
                    pltpu.sync_copy(rows_v.at[j % 2],
                                    tbl_sh.at[idxd_v.at[j]], add=True)
                return carry

            lax.fori_loop(0, ngrp, group, 0)
            plsc.subcore_barrier()
            pltpu.sync_copy(tbl_sh.at[pl.ds(r0, rpt)],
                            out_hbm.at[pl.ds(r0, rpt)])

        for ci in range(2):
            @pl.when(c == ci)
            def _(ci=ci):
                for p in range(spc):
                    one_slice(h_refs[ci * spc + p], out_refs[ci * spc + p])

    return pl.kernel(
        body,
        out_type=[jax.ShapeDtypeStruct((n_pad, w_core), F32)
                  for _ in range(n_slices)],
        mesh=_mesh(),
        compiler_params=_SC_PARAMS,
        scratch_types=[
            pltpu.VMEM((grp, e_chk), I32),
            pltpu.VMEM((grp, e_chk), I32),
            pltpu.VMEM((2, e_chk, w_core), F32),
            pltpu.VMEM((zrows, w_core), F32),
            pltpu.VMEM_SHARED((n_pad, w_core), F32),
            pltpu.SemaphoreType.DMA,
            pltpu.SemaphoreType.DMA,
        ],
    )


def _agg_full(n_pad, e_pad, e_chk, grp, zrows):
    rpt = n_pad // NTILE
    rows_per_w = e_pad // e_chk // 32
    ngrp = rows_per_w // grp
    assert rpt % zrows == 0 and ngrp * grp == rows_per_w

    def body(h_hbm, src_hbm, dst_hbm, out0, out1,
             idxs_v, idxd_v, rows_v, zer_v, tbl_sh, sem0, sem1):
        sems = (sem0, sem1)
        c = lax.axis_index("c")
        s = lax.axis_index("s")
        r0 = s * rpt
        row_base = (c * NTILE + s) * rows_per_w
        _zero_2d(zer_v, zrows, 128)
        for zi in range(rpt // zrows):
            pltpu.sync_copy(zer_v, tbl_sh.at[pl.ds(r0 + zi * zrows, zrows)])
        plsc.subcore_barrier()

        def group(gi, carry):
            c0 = row_base + gi * grp
            pltpu.sync_copy(src_hbm.at[pl.ds(c0, grp)], idxs_v)
            pltpu.sync_copy(dst_hbm.at[pl.ds(c0, grp)], idxd_v)
            cps = [pltpu.async_copy(h_hbm.at[idxs_v.at[0]],
                                    rows_v.at[0], sems[0])]
            for j in range(grp):
                if j + 1 < grp:
                    cps.append(pltpu.async_copy(
                        h_hbm.at[idxs_v.at[j + 1]],
                        rows_v.at[(j + 1) % 2], sems[(j + 1) % 2]))
                cps[j].wait()
                pltpu.sync_copy(rows_v.at[j % 2],
                                tbl_sh.at[idxd_v.at[j]], add=True)
            return carry

        lax.fori_loop(0, ngrp, group, 0)
        plsc.subcore_barrier()

        @pl.when(c == 0)
        def _():
            pltpu.sync_copy(tbl_sh.at[pl.ds(r0, rpt)],
                            out0.at[pl.ds(r0, rpt)])

        @pl.when(c == 1)
        def _():
            pltpu.sync_copy(tbl_sh.at[pl.ds(r0, rpt)],
                            out1.at[pl.ds(r0, rpt)])

    return pl.kernel(
        body,
        out_type=[jax.ShapeDtypeStruct((n_pad, 128), F32),
                  jax.ShapeDtypeStruct((n_pad, 128), F32)],
        mesh=_mesh(),
        compiler_params=_SC_PARAMS,
        scratch_types=[
            pltpu.VMEM((grp, e_chk), I32),
            pltpu.VMEM((grp, e_chk), I32),
            pltpu.VMEM((2, e_chk, 128), F32),
            pltpu.VMEM((zrows, 128), F32),
            pltpu.VMEM_SHARED((n_pad, 128), F32),
            pltpu.SemaphoreType.DMA,
            pltpu.SemaphoreType.DMA,
        ],
    )


def _pool_kernel(n_pad, r_chk):
    rpt = n_pad // 32
    nchk = rpt // r_chk
    brows = BP // NTILE
    assert nchk * r_chk == rpt

    def body(feat_hbm, gid_hbm, out0, out1, gid_v, rows_v, zer_v, tbl_sh):
        c = lax.axis_index("c")
        s = lax.axis_index("s")
        node0 = (c * NTILE + s) * rpt
        _zero_2d(zer_v, brows, 128)
        pltpu.sync_copy(zer_v, tbl_sh.at[pl.ds(s * brows, brows)])
        plsc.subcore_barrier()

        def chunk(i, carry):
            n0 = node0 + i * r_chk
            pltpu.sync_copy(gid_hbm.at[pl.ds(n0, r_chk)], gid_v)
            pltpu.sync_copy(feat_hbm.at[pl.ds(n0, r_chk)], rows_v)
            pltpu.sync_copy(rows_v, tbl_sh.at[gid_v], add=True)
            return carry

        lax.fori_loop(0, nchk, chunk, 0)
        plsc.subcore_barrier()

        @pl.when(c == 0)
        def _():
            pltpu.sync_copy(tbl_sh.at[pl.ds(s * brows, brows)],
                            out0.at[pl.ds(s * brows, brows)])

        @pl.when(c == 1)
        def _():
            pltpu.sync_copy(tbl_sh.at[pl.ds(s * brows, brows)],
                            out1.at[pl.ds(s * brows, brows)])

    return pl.kernel(
        body,
        out_type=[jax.ShapeDtypeStruct((BP, 128), F32),
                  jax.ShapeDtypeStruct((BP, 128), F32)],
        mesh=_mesh(),
        compiler_params=_SC_PARAMS,
        scratch_types=[
            pltpu.VMEM((r_chk,), I32),
            pltpu.VMEM((r_chk, 128), F32),
            pltpu.VMEM((brows, 128), F32),
            pltpu.VMEM_SHARED((BP, 128), F32),
        ],
    )


_BM = 512
_TC_PARAMS = pltpu.CompilerParams(dimension_semantics=("parallel",))


def _tc_scale_c(x, dsrc, ddst):
    n, w = x.shape

    def body(x_ref, ds_ref, dd_ref, x0, x1, x2, x3, ro_ref, ri_ref):
        ro = lax.rsqrt(jnp.maximum(ds_ref[...], 1.0))
        ri = lax.rsqrt(jnp.maximum(dd_ref[...], 1.0))
        xp = x_ref[...] * ro
        for i, o in enumerate((x0, x1, x2, x3)):
            o[...] = xp[:, i * 32:(i + 1) * 32]
        ro_ref[...] = ro
        ri_ref[...] = ri

    return pl.pallas_call(
        body,
        grid=(n // _BM,),
        compiler_params=_TC_PARAMS,
        in_specs=[
            pl.BlockSpec((_BM, w), lambda i: (i, 0)),
            pl.BlockSpec((_BM, 1), lambda i: (i, 0)),
            pl.BlockSpec((_BM, 1), lambda i: (i, 0)),
        ],
        out_specs=[pl.BlockSpec((_BM, 32), lambda i: (i, 0))] * 4 + [
            pl.BlockSpec((_BM, 1), lambda i: (i, 0)),
            pl.BlockSpec((_BM, 1), lambda i: (i, 0)),
        ],
        out_shape=[jax.ShapeDtypeStruct((n, 32), F32)] * 4 + [
            jax.ShapeDtypeStruct((n, 1), F32),
            jax.ShapeDtypeStruct((n, 1), F32),
        ],
    )(x, dsrc, ddst)


def _tc_scale_p(x, dsrc, ddst):
    n, w = x.shape

    def body(x_ref, ds_ref, dd_ref, xp_ref, ro_ref, ri_ref):
        ro = lax.rsqrt(jnp.maximum(ds_ref[...], 1.0))
        ri = lax.rsqrt(jnp.maximum(dd_ref[...], 1.0))
        xp_ref[...] = x_ref[...] * ro
        ro_ref[...] = ro
        ri_ref[...] = ri

    return pl.pallas_call(
        body,
        grid=(n // _BM,),
        compiler_params=_TC_PARAMS,
        in_specs=[
            pl.BlockSpec((_BM, w), lambda i: (i, 0)),
            pl.BlockSpec((_BM, 1), lambda i: (i, 0)),
            pl.BlockSpec((_BM, 1), lambda i: (i, 0)),
        ],
        out_specs=[
            pl.BlockSpec((_BM, w), lambda i: (i, 0)),
            pl.BlockSpec((_BM, 1), lambda i: (i, 0)),
            pl.BlockSpec((_BM, 1), lambda i: (i, 0)),
        ],
        out_shape=[
            jax.ShapeDtypeStruct((n, w), F32),
            jax.ShapeDtypeStruct((n, 1), F32),
            jax.ShapeDtypeStruct((n, 1), F32),
        ],
    )(x, dsrc, ddst)


def _tc_mid_c(a_slices, ri, ro, w1, b1, w2):
    n = a_slices[0].shape[0]

    def body(a0, a1, a2, a3, ri_ref, ro_ref, w1_ref, b1_ref, w2_ref,
             o0, o1, o2, o3):
        t = jnp.dot(a0[...], w1_ref[0:32, :], preferred_element_type=F32)
        t += jnp.dot(a1[...], w1_ref[32:64, :], preferred_element_type=F32)
        t += jnp.dot(a2[...], w1_ref[64:96, :], preferred_element_type=F32)
        t += jnp.dot(a3[...], w1_ref[96:128, :], preferred_element_type=F32)
        t = jnp.maximum(t * ri_ref[...] + b1_ref[...], 0.0)
        h2 = jnp.dot(t * ro_ref[...], w2_ref[...], preferred_element_type=F32)
        for i, o in enumerate((o0, o1, o2, o3)):
            o[...] = h2[:, i * 32:(i + 1) * 32]

    return pl.pallas_call(
        body,
        grid=(n // _BM,),
        compiler_params=_TC_PARAMS,
        in_specs=[pl.BlockSpec((_BM, 32), lambda i: (i, 0))] * 4 + [
            pl.BlockSpec((_BM, 1), lambda i: (i, 0)),
            pl.BlockSpec((_BM, 1), lambda i: (i, 0)),
            pl.BlockSpec((HID, HID), lambda i: (0, 0)),
            pl.BlockSpec((1, HID), lambda i: (0, 0)),
            pl.BlockSpec((HID, HID), lambda i: (0, 0)),
        ],
        out_specs=[pl.BlockSpec((_BM, 32), lambda i: (i, 0))] * 4,
        out_shape=[jax.ShapeDtypeStruct((n, 32), F32)] * 4,
    )(*a_slices, ri, ro, w1, b1, w2)


def _tc_mid_p(a0, a1, ri, ro, w1, b1, w2):
    n = a0.shape[0]

    def body(a0_ref, a1_ref, ri_ref, ro_ref, w1_ref, b1_ref, w2_ref, out):
        a = a0_ref[...] + a1_ref[...]
        t = jnp.dot(a, w1_ref[...], preferred_element_type=F32)
        t = jnp.maximum(t * ri_ref[...] + b1_ref[...], 0.0)
        out[...] = jnp.dot(t * ro_ref[...], w2_ref[...],
                           preferred_element_type=F32)

    return pl.pallas_call(
        body,
        grid=(n // _BM,),
        compiler_params=_TC_PARAMS,
        in_specs=[
            pl.BlockSpec((_BM, HID), lambda i: (i, 0)),
            pl.BlockSpec((_BM, HID), lambda i: (i, 0)),
            pl.BlockSpec((_BM, 1), lambda i: (i, 0)),
            pl.BlockSpec((_BM, 1), lambda i: (i, 0)),
            pl.BlockSpec((HID, HID), lambda i: (0, 0)),
            pl.BlockSpec((1, HID), lambda i: (0, 0)),
            pl.BlockSpec((HID, HID), lambda i: (0, 0)),
        ],
        out_specs=pl.BlockSpec((_BM, HID), lambda i: (i, 0)),
        out_shape=jax.ShapeDtypeStruct((n, HID), F32),
    )(a0, a1, ri, ro, w1, b1, w2)


def _tc_post_c(gs, ri, b2):
    n = gs[0].shape[0]

    def body(g0, g1, g2, g3, ri_ref, b_ref, out):
        cf = jnp.concatenate([g0[...], g1[...], g2[...], g3[...]], axis=1)
        out[...] = jnp.maximum(cf * ri_ref[...] + b_ref[...], 0.0)

    return pl.pallas_call(
        body,
        grid=(n // _BM,),
        compiler_params=_TC_PARAMS,
        in_specs=[pl.BlockSpec((_BM, 32), lambda i: (i, 0))] * 4 + [
            pl.BlockSpec((_BM, 1), lambda i: (i, 0)),
            pl.BlockSpec((1, HID), lambda i: (0, 0)),
        ],
        out_specs=pl.BlockSpec((_BM, HID), lambda i: (i, 0)),
        out_shape=jax.ShapeDtypeStruct((n, HID), F32),
    )(*gs, ri, b2)


def _tc_post_p(g0, g1, ri, b2):
    n = g0.shape[0]

    def body(g0_ref, g1_ref, ri_ref, b_ref, out):
        out[...] = jnp.maximum(
            (g0_ref[...] + g1_ref[...]) * ri_ref[...] + b_ref[...], 0.0)

    return pl.pallas_call(
        body,
        grid=(n // _BM,),
        compiler_params=_TC_PARAMS,
        in_specs=[
            pl.BlockSpec((_BM, HID), lambda i: (i, 0)),
            pl.BlockSpec((_BM, HID), lambda i: (i, 0)),
            pl.BlockSpec((_BM, 1), lambda i: (i, 0)),
            pl.BlockSpec((1, HID), lambda i: (0, 0)),
        ],
        out_specs=pl.BlockSpec((_BM, HID), lambda i: (i, 0)),
        out_shape=jax.ShapeDtypeStruct((n, HID), F32),
    )(g0, g1, ri, b2)


def _tc_final(p0, p1, q0, q1, cc, cp, w1a, w1b, b1, w2r, b2):
    def body(p0_ref, p1_ref, q0_ref, q1_ref, cc_ref, cp_ref,
             w1a_ref, w1b_ref, b1_ref, w2_ref, b2_ref, out_ref):
        mc = (p0_ref[...] + p1_ref[...]) / jnp.maximum(cc_ref[...], 1.0)
        mp = (q0_ref[...] + q1_ref[...]) / jnp.maximum(cp_ref[...], 1.0)
        h = jnp.dot(mc, w1a_ref[...], preferred_element_type=F32)
        h += jnp.dot(mp, w1b_ref[...], preferred_element_type=F32)
        h = jnp.maximum(h + b1_ref[...], 0.0)
        out_ref[...] = jnp.sum(h * w2_ref[...], axis=1, keepdims=True) \
            + b2_ref[...]

    return pl.pallas_call(
        body,
        out_shape=jax.ShapeDtypeStruct((NG, 1), F32),
    )(p0, p1, q0, q1, cc, cp, w1a, w1b, b1, w2r, b2)


def kernel(compound_x, compound_edge_index, compound_graph_ids,
           protein_x, protein_edge_index, protein_graph_ids,
           Wc1, bc1, Wc2, bc2, Wp1, bp1, Wp2, bp2, Wm1, bm1, Wm2, bm2):
    dump_c = jnp.full((ECP - EC,), NCP - 8, I32)
    dump_p = jnp.full((EPP - EP,), NPP - 8, I32)

    xc = jnp.pad(compound_x, ((0, NCP - NC), (0, HID - C_IN)))
    src_c = jnp.concatenate([compound_edge_index[0], dump_c])
    dst_c = jnp.concatenate([compound_edge_index[1], dump_c])
    gid_c = jnp.concatenate(
        [compound_graph_ids, jnp.full((NCP - NC,), NG, I32)])

    xp = jnp.pad(protein_x, ((0, NPP - NP2), (0, 0)))
    src_p = jnp.concatenate([protein_edge_index[0], dump_p])
    dst_p = jnp.concatenate([protein_edge_index[1], dump_p])
    gid_p = jnp.concatenate(
        [protein_graph_ids, jnp.full((NPP - NP2,), NG, I32)])

    iota = jnp.arange(RC, dtype=I32)

    src_c2 = src_c.reshape(-1, 392)
    dst_c2 = dst_c.reshape(-1, 392)
    src_p2 = src_p.reshape(-1, 160)
    dst_p2 = dst_p.reshape(-1, 160)

    dsc, ddc, dsp, ddp, cc, cp = _deg_kernel()(
        src_c, dst_c, src_p, dst_p, gid_c, gid_p, iota)
    dsc = dsc.reshape(-1, 1)
    ddc = ddc.reshape(-1, 1)
    dsp = dsp.reshape(-1, 1)
    ddp = ddp.reshape(-1, 1)
    cc = cc.reshape(-1, 1)[:NG]
    cp = cp.reshape(-1, 1)[:NG]

    w1c = jnp.pad(Wc1, ((0, HID - C_IN), (0, 0)))
    b1c = bc1.reshape(1, HID)
    b2c = bc2.reshape(1, HID)
    b1p = bp1.reshape(1, HID)
    b2p = bp2.reshape(1, HID)

    x0, x1, x2, x3, ro_c, ri_c = _tc_scale_c(xc, dsc, ddc)
    agg_c1 = _agg_slices(NCP, 32, 4, ECP, 392, 4, 28)
    a0, a1, a2, a3 = agg_c1(x0, x1, x2, x3, src_c2, dst_c2)
    h0, h1, h2, h3 = _tc_mid_c((a0, a1, a2, a3), ri_c, ro_c, w1c, b1c, Wc2)
    agg_c2 = _agg_slices(NCP, 32, 4, ECP, 392, 4, 28)
    g0, g1, g2, g3 = agg_c2(h0, h1, h2, h3, src_c2, dst_c2)
    cf = _tc_post_c((g0, g1, g2, g3), ri_c, b2c)
    p0, p1 = _pool_kernel(NCP, 224)(cf, gid_c)

    yp, ro_p, ri_p = _tc_scale_p(xp, dsp, ddp)
    b0, b1_ = _agg_full(NPP, EPP, 160, 4, 20)(yp, src_p2, dst_p2)
    h2p = _tc_mid_p(b0, b1_, ri_p, ro_p, Wp1, b1p, Wp2)
    gp0, gp1 = _agg_full(NPP, EPP, 160, 4, 20)(h2p, src_p2, dst_p2)
    pf = _tc_post_p(gp0, gp1, ri_p, b2p)
    q0, q1 = _pool_kernel(NPP, 320)(pf, gid_p)

    out = _tc_final(p0[:NG], p1[:NG], q0[:NG], q1[:NG], cc, cp,
                    Wm1[:HID], Wm1[HID:], bm1.reshape(1, HID),
                    Wm2.reshape(1, HID), bm2.reshape(1, 1))
    return jnp.squeeze(out, axis=-1)
